# baseline (speedup 1.0000x reference)
import functools

import jax
import jax.numpy as jnp
from jax import lax
from jax.experimental import pallas as pl
from jax.experimental.pallas import tpu as pltpu
from jax.experimental.pallas import tpu_sc as plsc

HIDDEN = 64
CHUNK = 128
KMAX = 6

NC = 2
NS = 16
NW = NC * NS


def _plan(n_rows):
    n_chunks = -(-n_rows // CHUNK)
    cpw = -(-n_chunks // NW)
    best = None
    for k in range(1, KMAX + 1):
        gw = -(-cpw // k)
        gw += gw % 2
        span = gw * k
        cost = span * 1.2 + gw * 2.0
        if best is None or cost < best[0]:
            best = (cost, k, gw)
    _, k, gw = best
    return k, gw


def _task_gather(table, idx_hbm, out_hbm, K, gw, wid,
                 idx_v, rows0, rows1, sem_g, sem_w0, sem_w1):
    span = gw * K
    grows = K * CHUNK
    c0 = wid * span
    hb = gw // 2

    def wr_desc(rows_v, sem, goff):
        return pltpu.make_async_copy(
            rows_v.at[pl.ds(0, grows)],
            out_hbm.at[pl.ds(goff * CHUNK, grows)],
            sem)

    def run_group(h, rows_v, sem_w, parity, idx_base):
        g = h * 2 + parity
        goff = c0 + g * K

        @pl.when(h >= 1)
        def _():
            wr_desc(rows_v, sem_w, goff - 2 * K).wait()

        for b in range(K):
            pltpu.make_async_copy(
                table.at[idx_v.at[idx_base + b]],
                rows_v.at[pl.ds(b * CHUNK, CHUNK)],
                sem_g).start()
        for b in range(K):
            pltpu.make_async_copy(
                table.at[idx_v.at[idx_base + b]],
                rows_v.at[pl.ds(b * CHUNK, CHUNK)],
                sem_g).wait()
        wr_desc(rows_v, sem_w, goff).start()

    def step(h, _):
        pltpu.sync_copy(
            idx_hbm.at[pl.ds(c0 + h * 2 * K, 2 * K)],
            idx_v.at[pl.ds(0, 2 * K)])
        run_group(h, rows0, sem_w0, 0, 0)
        run_group(h, rows1, sem_w1, 1, K)
        return ()

    lax.fori_loop(0, hb, step, ())

    wr_desc(rows0, sem_w0, c0 + (gw - 2) * K).wait()
    wr_desc(rows1, sem_w1, c0 + (gw - 1) * K).wait()


@functools.lru_cache(maxsize=None)
def _make_sc_gather(task_descs):
    mesh = plsc.VectorSubcoreMesh(core_axis_name="c", subcore_axis_name="s")
    nt = len(task_descs)

    def body(*refs):
        tables = refs[0:3]
        idxs = refs[3:3 + nt]
        outs = refs[3 + nt:3 + 2 * nt]
        idx_v, rows0, rows1, sem_g, sem_w0, sem_w1 = refs[3 + 2 * nt:]
        wid = lax.axis_index("s") * NC + lax.axis_index("c")
        for i, (tslot, _, K, gw) in enumerate(task_descs):
            _task_gather(tables[tslot], idxs[i], outs[i], K, gw, wid,
                         idx_v, rows0, rows1, sem_g, sem_w0, sem_w1)

    return pl.kernel(
        body,
        out_type=tuple(
            jax.ShapeDtypeStruct((npad, HIDDEN), jnp.float32)
            for (_, npad, _, _) in task_descs),
        mesh=mesh,
        compiler_params=pltpu.CompilerParams(use_tc_tiling_on_sc=False),
        scratch_types=[
            pltpu.VMEM((2 * KMAX, CHUNK), jnp.int32),
            pltpu.VMEM((KMAX * CHUNK, HIDDEN), jnp.float32),
            pltpu.VMEM((KMAX * CHUNK, HIDDEN), jnp.float32),
            pltpu.SemaphoreType.DMA,
            pltpu.SemaphoreType.DMA,
            pltpu.SemaphoreType.DMA,
        ],
    )


_TSLOTS = (0, 1, 2, 1, 1, 2)
_GROUPS = ((0, 2, 5), (1, 4), (3,))


def kernel(user, traj, geo, long_traj, traj_graph_x, geo_graph_x,
           user_table, loc_table, geo_table):
    tables = (user_table, loc_table, geo_table)
    srcs = (user, traj, geo, long_traj, traj_graph_x, geo_graph_x)

    descs = []
    idx_arrays = []
    for src, tslot in zip(srcs, _TSLOTS):
        flat = src.reshape(-1).astype(jnp.int32)
        n = flat.shape[0]
        K, gw = _plan(n)
        npad = NW * gw * K * CHUNK
        if npad > n:
            pad = jnp.arange(npad - n, dtype=jnp.int32) % tables[tslot].shape[0]
            flat = jnp.concatenate([flat, pad])
        descs.append((tslot, npad, K, gw))
        idx_arrays.append(flat.reshape(-1, CHUNK))

    one = user_table[0, 0] * jnp.float32(0.0) + jnp.float32(1.0)

    outs = [None] * len(srcs)
    for group in _GROUPS:
        gdescs = tuple(descs[i] for i in group)
        gouts = _make_sc_gather(gdescs)(
            *tables, *[idx_arrays[i] for i in group])
        for i, o in zip(group, gouts):
            src = srcs[i]
            npad = o.shape[0]
            n = 1
            for d in src.shape:
                n *= d
            o = lax.optimization_barrier(o.reshape(npad // 2, 2 * HIDDEN))
            o = o.reshape(npad, HIDDEN)[:n].reshape(src.shape + (HIDDEN,))
            outs[i] = o * one

    return tuple(outs)


# speedup vs baseline: 1.0553x; 1.0553x over previous
import functools

import jax
import jax.numpy as jnp
from jax import lax
from jax.experimental import pallas as pl
from jax.experimental.pallas import tpu as pltpu
from jax.experimental.pallas import tpu_sc as plsc

HIDDEN = 64
CHUNK = 128
KMAX = 6

NC = 2
NS = 16
NW = NC * NS


def _plan(n_rows):
    n_chunks = -(-n_rows // CHUNK)
    cpw = -(-n_chunks // NW)
    best = None
    for k in range(1, KMAX + 1):
        gw = -(-cpw // k)
        gw += gw % 2
        span = gw * k
        cost = span * 1.2 + gw * 2.0
        if best is None or cost < best[0]:
            best = (cost, k, gw)
    _, k, gw = best
    return k, gw


@functools.lru_cache(maxsize=None)
def _make_sc_gather(n_pad_rows, K, gw):
    mesh = plsc.VectorSubcoreMesh(core_axis_name="c", subcore_axis_name="s")

    def body(table, idx_hbm, out_hbm, idx_v, rows0, rows1,
             sem_g, sem_w0, sem_w1):
        wid = lax.axis_index("s") * NC + lax.axis_index("c")
        span = gw * K
        grows = K * CHUNK
        c0 = wid * span
        hb = gw // 2

        def wr_desc(rows_v, sem, goff):
            return pltpu.make_async_copy(
                rows_v.at[pl.ds(0, grows)],
                out_hbm.at[pl.ds(goff * CHUNK, grows)],
                sem)

        def run_group(h, rows_v, sem_w, parity, idx_base):
            g = h * 2 + parity
            goff = c0 + g * K

            @pl.when(h >= 1)
            def _():
                wr_desc(rows_v, sem_w, goff - 2 * K).wait()

            for b in range(K):
                pltpu.make_async_copy(
                    table.at[idx_v.at[idx_base + b]],
                    rows_v.at[pl.ds(b * CHUNK, CHUNK)],
                    sem_g).start()
            for b in range(K):
                pltpu.make_async_copy(
                    table.at[idx_v.at[idx_base + b]],
                    rows_v.at[pl.ds(b * CHUNK, CHUNK)],
                    sem_g).wait()
            wr_desc(rows_v, sem_w, goff).start()

        def step(h, _):
            pltpu.sync_copy(
                idx_hbm.at[pl.ds(c0 + h * 2 * K, 2 * K)],
                idx_v.at[pl.ds(0, 2 * K)])
            run_group(h, rows0, sem_w0, 0, 0)
            run_group(h, rows1, sem_w1, 1, K)
            return ()

        lax.fori_loop(0, hb, step, ())

        wr_desc(rows0, sem_w0, c0 + (gw - 2) * K).wait()
        wr_desc(rows1, sem_w1, c0 + (gw - 1) * K).wait()

    return pl.kernel(
        body,
        out_type=jax.ShapeDtypeStruct((n_pad_rows, HIDDEN), jnp.float32),
        mesh=mesh,
        compiler_params=pltpu.CompilerParams(use_tc_tiling_on_sc=False),
        scratch_types=[
            pltpu.VMEM((2 * KMAX, CHUNK), jnp.int32),
            pltpu.VMEM((KMAX * CHUNK, HIDDEN), jnp.float32),
            pltpu.VMEM((KMAX * CHUNK, HIDDEN), jnp.float32),
            pltpu.SemaphoreType.DMA,
            pltpu.SemaphoreType.DMA,
            pltpu.SemaphoreType.DMA,
        ],
    )


def kernel(user, traj, geo, long_traj, traj_graph_x, geo_graph_x,
           user_table, loc_table, geo_table):
    tables = (user_table, loc_table, geo_table)
    srcs = (user, traj, geo, long_traj, traj_graph_x, geo_graph_x)
    tslots = (0, 1, 2, 1, 1, 2)

    order = (0, 2, 5, 1, 3, 4)

    outs = [None] * len(srcs)
    for t in order:
        src, tslot = srcs[t], tslots[t]
        table = tables[tslot]
        flat = src.reshape(-1).astype(jnp.int32)
        n = flat.shape[0]
        K, gw = _plan(n)
        npad = NW * gw * K * CHUNK
        if npad > n:
            pad = jnp.arange(npad - n, dtype=jnp.int32) % table.shape[0]
            flat = jnp.concatenate([flat, pad])
        idx2d = flat.reshape(-1, CHUNK)

        o = _make_sc_gather(npad, K, gw)(table, idx2d)
        o = lax.optimization_barrier(o.reshape(npad // 2, 2 * HIDDEN))
        outs[t] = o.reshape(npad, HIDDEN)[:n].reshape(src.shape + (HIDDEN,))

    return tuple(outs)
